# baseline (speedup 1.0000x reference)
import functools

import jax
import jax.numpy as jnp
from jax import lax
from jax.experimental import pallas as pl
from jax.experimental.pallas import tpu as pltpu
from jax.experimental.pallas import tpu_sc as plsc

_D = 128
_TPAD = 904
_NC, _NS = 2, 16
_NW = _NC * _NS
_CH = 32
_GRP = 5
_NSEG = 40


def _table_body(row_ref, col_ref, out_ref):
    t = lax.broadcasted_iota(jnp.int32, (_TPAD, 32), 0)
    k = lax.broadcasted_iota(jnp.int32, (_TPAD, 32), 1)
    r = jnp.clip(t // 30, 0, 29)
    c = t - 30 * (t // 30)
    oh_r = (r == k).astype(jnp.float32)
    oh_c = (c == k).astype(jnp.float32)
    row_emb = jnp.dot(oh_r, row_ref[...], preferred_element_type=jnp.float32,
                      precision=lax.Precision.HIGHEST)
    col_emb = jnp.dot(oh_c, col_ref[...], preferred_element_type=jnp.float32,
                      precision=lax.Precision.HIGHEST)
    out_ref[...] = jnp.concatenate([row_emb, col_emb], axis=-1)


_build_table = pl.pallas_call(
    _table_body,
    out_shape=jax.ShapeDtypeStruct((_TPAD, _D), jnp.float32),
)


@functools.lru_cache(maxsize=None)
def _make_gather(B):
    bpw = B // _NW
    seg = bpw // _NSEG
    cps = seg // _CH
    gps = cps // _GRP
    mesh = plsc.VectorSubcoreMesh(core_axis_name="c", subcore_axis_name="s")

    @functools.partial(
        pl.kernel,
        mesh=mesh,
        out_type=jax.ShapeDtypeStruct((B, _D), jnp.float32),
        compiler_params=pltpu.CompilerParams(needs_layout_passes=False),
        scratch_types=[
            pltpu.VMEM((seg,), jnp.int32),
            pltpu.VMEM((seg,), jnp.int32),
            pltpu.VMEM((_TPAD, _D), jnp.float32),
            pltpu.VMEM((_CH, _D), jnp.float32),
            pltpu.VMEM((_CH, _D), jnp.float32),
            pltpu.VMEM((_CH, _D), jnp.float32),
            pltpu.SemaphoreType.DMA,
            pltpu.SemaphoreType.DMA,
            pltpu.SemaphoreType.DMA,
            pltpu.SemaphoreType.DMA,
            pltpu.SemaphoreType.DMA,
            pltpu.SemaphoreType.DMA,
        ],
    )
    def gather(table_hbm, idx_hbm, out_hbm,
               idxb0, idxb1, table_t, bufv0, bufv1, bufe,
               isem0, isem1, gsem, sv0, sv1, se):
        wid = lax.axis_index("s") * _NC + lax.axis_index("c")
        base = wid * bpw

        pltpu.sync_copy(table_hbm, table_t)

        cols = [lax.broadcasted_iota(jnp.int32, (16,), 0) + 16 * j
                for j in range(8)]

        def i_start(s, ib, isem):
            pltpu.make_async_copy(idx_hbm.at[wid, s], ib, isem).start()

        def i_wait(ib, isem):
            pltpu.make_async_copy(idx_hbm.at[wid, 0], ib, isem).wait()

        def g_start(cp, ib):
            pltpu.make_async_copy(
                table_hbm.at[ib.at[pl.ds(cp * _CH, _CH)]], bufe, gsem).start()

        def g_wait():
            pltpu.make_async_copy(
                table_hbm.at[idxb0.at[pl.ds(0, _CH)]], bufe, gsem).wait()

        def st_start(c, buf, sem):
            pltpu.make_async_copy(
                buf, out_hbm.at[pl.ds(base + c * _CH, _CH)], sem).start()

        def st_wait(buf, sem):
            pltpu.make_async_copy(
                buf, out_hbm.at[pl.ds(base, _CH)], sem).wait()

        def produce(cp, ib, buf):
            @plsc.parallel_loop(0, _CH, 16, unroll=2)
            def _(rb):
                pv = ib[pl.ds(cp * _CH + rb, 16)]
                for r in range(16):
                    prow = jnp.take_along_axis(
                        pv, jnp.full((16,), r, jnp.int32), axis=0)
                    for j in range(8):
                        v = plsc.load_gather(table_t, [prow, cols[j]])
                        buf[rb + r, pl.ds(16 * j, 16)] = v

        i_start(0, idxb0, isem0)

        def seg_half(ii, ib, isem, nib, nisem):
            i_wait(ib, isem)

            @pl.when(ii + 1 < _NSEG)
            def _():
                i_start(ii + 1, nib, nisem)

            def grp_body(g, carry):
                cbase = ii * cps + g * _GRP

                @pl.when(g > 0)
                def _():
                    g_wait()
                    st_start(cbase - 1, bufe, se)

                for k in range(_GRP - 1):
                    buf, sv = (bufv0, sv0) if k % 2 == 0 else (bufv1, sv1)
                    if k < 2:
                        @pl.when(ii * gps + g > 0)
                        def _(buf=buf, sv=sv):
                            st_wait(buf, sv)
                    else:
                        st_wait(buf, sv)
                    produce(g * _GRP + k, ib, buf)
                    st_start(cbase + k, buf, sv)

                @pl.when(ii * gps + g > 0)
                def _():
                    st_wait(bufe, se)

                g_start(g * _GRP + _GRP - 1, ib)

                @pl.when(g == gps - 1)
                def _():
                    g_wait()
                    st_start(cbase + _GRP - 1, bufe, se)

                return carry

            lax.fori_loop(0, gps, grp_body, 0)

        def seg_body(jj, carry):
            seg_half(jj * 2, idxb0, isem0, idxb1, isem1)
            seg_half(jj * 2 + 1, idxb1, isem1, idxb0, isem0)
            return carry

        lax.fori_loop(0, _NSEG // 2, seg_body, 0)
        st_wait(bufv0, sv0)
        st_wait(bufv1, sv1)
        st_wait(bufe, se)

    return gather


def kernel(positions, row_table, col_table):
    s0, s1 = positions.shape
    b = s0 * s1
    row_p = jnp.pad(row_table.astype(jnp.float32), ((0, 2), (0, 0)))
    col_p = jnp.pad(col_table.astype(jnp.float32), ((0, 2), (0, 0)))
    table = _build_table(row_p, col_p)
    idx = positions.astype(jnp.int32).reshape(_NW, _NSEG, b // (_NW * _NSEG))
    out = _make_gather(b)(table, idx)
    return out.reshape(s0, s1, _D)


# speedup vs baseline: 2.8392x; 2.8392x over previous
import functools

import jax
import jax.numpy as jnp
from jax import lax
from jax.experimental import pallas as pl
from jax.experimental.pallas import tpu as pltpu
from jax.experimental.pallas import tpu_sc as plsc

_D = 128
_TPAD = 904
_NC, _NS = 2, 16
_NW = _NC * _NS
_CH = 32
_GRP = 5
_NSEG = 40


def _table_body(row_ref, col_ref, out_ref):
    t = lax.broadcasted_iota(jnp.int32, (_TPAD, 32), 0)
    k = lax.broadcasted_iota(jnp.int32, (_TPAD, 32), 1)
    r = jnp.clip(t // 30, 0, 29)
    c = t - 30 * (t // 30)
    oh_r = (r == k).astype(jnp.float32)
    oh_c = (c == k).astype(jnp.float32)
    row_emb = jnp.dot(oh_r, row_ref[...], preferred_element_type=jnp.float32,
                      precision=lax.Precision.HIGHEST)
    col_emb = jnp.dot(oh_c, col_ref[...], preferred_element_type=jnp.float32,
                      precision=lax.Precision.HIGHEST)
    out_ref[...] = jnp.concatenate([row_emb, col_emb], axis=-1)


_build_table = pl.pallas_call(
    _table_body,
    out_shape=jax.ShapeDtypeStruct((_TPAD, _D), jnp.float32),
)


@functools.lru_cache(maxsize=None)
def _make_gather(B):
    bpw = B // _NW
    seg = bpw // _NSEG
    cps = seg // _CH
    gps = cps // _GRP
    mesh = plsc.VectorSubcoreMesh(core_axis_name="c", subcore_axis_name="s")

    @functools.partial(
        pl.kernel,
        mesh=mesh,
        out_type=jax.ShapeDtypeStruct((B, _D), jnp.float32),
        compiler_params=pltpu.CompilerParams(needs_layout_passes=False),
        scratch_types=[
            pltpu.VMEM((seg,), jnp.int32),
            pltpu.VMEM((seg,), jnp.int32),
            pltpu.VMEM((_TPAD, _D), jnp.float32),
            pltpu.VMEM((_CH, _D), jnp.float32),
            pltpu.VMEM((_CH, _D), jnp.float32),
            pltpu.VMEM((_CH, _D), jnp.float32),
            pltpu.SemaphoreType.DMA,
            pltpu.SemaphoreType.DMA,
            pltpu.SemaphoreType.DMA,
            pltpu.SemaphoreType.DMA,
            pltpu.SemaphoreType.DMA,
            pltpu.SemaphoreType.DMA,
        ],
    )
    def gather(table_hbm, idx_hbm, out_hbm,
               idxb0, idxb1, table_t, bufv0, bufv1, bufe,
               isem0, isem1, gsem, sv0, sv1, se):
        wid = lax.axis_index("s") * _NC + lax.axis_index("c")
        base = wid * bpw

        pltpu.sync_copy(table_hbm, table_t)

        cols = [lax.broadcasted_iota(jnp.int32, (16,), 0) + 16 * j
                for j in range(8)]

        def i_start(s, ib, isem):
            pltpu.make_async_copy(idx_hbm.at[wid, s], ib, isem).start()

        def i_wait(ib, isem):
            pltpu.make_async_copy(idx_hbm.at[wid, 0], ib, isem).wait()

        def g_start(cp, ib):
            pltpu.make_async_copy(
                table_hbm.at[ib.at[pl.ds(cp * _CH, _CH)]], bufe, gsem).start()

        def g_wait():
            pltpu.make_async_copy(
                table_hbm.at[idxb0.at[pl.ds(0, _CH)]], bufe, gsem).wait()

        def st_start(c, buf, sem):
            pltpu.make_async_copy(
                buf, out_hbm.at[pl.ds(base + c * _CH, _CH)], sem).start()

        def st_wait(buf, sem):
            pltpu.make_async_copy(
                buf, out_hbm.at[pl.ds(base, _CH)], sem).wait()

        def produce(cp, ib, buf):
            @plsc.parallel_loop(0, _CH, 1, unroll=16)
            def _(r):
                r16 = (r // 16) * 16
                pv = ib[pl.ds(cp * _CH + r16, 16)]
                lane = jnp.broadcast_to(r - r16, (16,))
                prow = jnp.take_along_axis(pv, lane, axis=0)
                for j in range(8):
                    v = plsc.load_gather(table_t, [prow, cols[j]])
                    buf[r, pl.ds(16 * j, 16)] = v

        i_start(0, idxb0, isem0)

        def seg_half(ii, ib, isem, nib, nisem):
            i_wait(ib, isem)

            @pl.when(ii + 1 < _NSEG)
            def _():
                i_start(ii + 1, nib, nisem)

            def grp_body(g, carry):
                cbase = ii * cps + g * _GRP

                @pl.when(g > 0)
                def _():
                    g_wait()
                    st_start(cbase - 1, bufe, se)

                for k in range(_GRP - 1):
                    buf, sv = (bufv0, sv0) if k % 2 == 0 else (bufv1, sv1)
                    if k < 2:
                        @pl.when(ii * gps + g > 0)
                        def _(buf=buf, sv=sv):
                            st_wait(buf, sv)
                    else:
                        st_wait(buf, sv)
                    produce(g * _GRP + k, ib, buf)
                    st_start(cbase + k, buf, sv)

                @pl.when(ii * gps + g > 0)
                def _():
                    st_wait(bufe, se)

                g_start(g * _GRP + _GRP - 1, ib)

                @pl.when(g == gps - 1)
                def _():
                    g_wait()
                    st_start(cbase + _GRP - 1, bufe, se)

                return carry

            lax.fori_loop(0, gps, grp_body, 0)

        def seg_body(jj, carry):
            seg_half(jj * 2, idxb0, isem0, idxb1, isem1)
            seg_half(jj * 2 + 1, idxb1, isem1, idxb0, isem0)
            return carry

        lax.fori_loop(0, _NSEG // 2, seg_body, 0)
        st_wait(bufv0, sv0)
        st_wait(bufv1, sv1)
        st_wait(bufe, se)

    return gather


def kernel(positions, row_table, col_table):
    s0, s1 = positions.shape
    b = s0 * s1
    row_p = jnp.pad(row_table.astype(jnp.float32), ((0, 2), (0, 0)))
    col_p = jnp.pad(col_table.astype(jnp.float32), ((0, 2), (0, 0)))
    table = _build_table(row_p, col_p)
    idx = positions.astype(jnp.int32).reshape(_NW, _NSEG, b // (_NW * _NSEG))
    out = _make_gather(b)(table, idx)
    return out.reshape(s0, s1, _D)


# speedup vs baseline: 5.2428x; 1.8465x over previous
import functools

import jax
import jax.numpy as jnp
from jax import lax
from jax.experimental import pallas as pl
from jax.experimental.pallas import tpu as pltpu
from jax.experimental.pallas import tpu_sc as plsc

_D = 128
_HALF = _D // 2
_TPAD = 1024
_NC, _NS = 2, 16
_NW = _NC * _NS
_C = 128


def _table_body(row_ref, col_ref, out_ref):
    t = lax.broadcasted_iota(jnp.int32, (_TPAD, 32), 0)
    k = lax.broadcasted_iota(jnp.int32, (_TPAD, 32), 1)
    r = jnp.clip(t // 30, 0, 29)
    c = t - 30 * (t // 30)
    oh_r = (r == k).astype(jnp.float32)
    oh_c = (c == k).astype(jnp.float32)
    row_emb = jnp.dot(oh_r, row_ref[...], preferred_element_type=jnp.float32,
                      precision=lax.Precision.HIGHEST)
    col_emb = jnp.dot(oh_c, col_ref[...], preferred_element_type=jnp.float32,
                      precision=lax.Precision.HIGHEST)
    out_ref[...] = jnp.concatenate([row_emb, col_emb], axis=-1)


_build_table = pl.pallas_call(
    _table_body,
    out_shape=jax.ShapeDtypeStruct((_TPAD, _D), jnp.float32),
)


@functools.lru_cache(maxsize=None)
def _make_gather(B):
    npw = B // (_NW * _C)
    mesh = plsc.VectorSubcoreMesh(core_axis_name="c", subcore_axis_name="s")

    @functools.partial(
        pl.kernel,
        mesh=mesh,
        out_type=jax.ShapeDtypeStruct((B, _D), jnp.float32),
        scratch_types=[
            pltpu.VMEM((npw, _C), jnp.int32),
            pltpu.VMEM((4, _C, _D), jnp.float32),
            pltpu.VMEM_SHARED((_TPAD, _D), jnp.float32),
            pltpu.SemaphoreType.DMA,
            pltpu.SemaphoreType.DMA,
            pltpu.SemaphoreType.DMA,
            pltpu.SemaphoreType.DMA,
            pltpu.SemaphoreType.DMA,
            pltpu.SemaphoreType.DMA,
            pltpu.SemaphoreType.DMA,
            pltpu.SemaphoreType.DMA,
        ],
    )
    def gather(table_hbm, idx_hbm, out_hbm, idx_v, rows, table_sh,
               g0, g1, g2, g3, s0, s1, s2, s3):
        gsem = (g0, g1, g2, g3)
        ssem = (s0, s1, s2, s3)
        wid = lax.axis_index("s") * _NC + lax.axis_index("c")
        base = wid * (npw * _C)

        @pl.when(lax.axis_index("s") == 0)
        def _():
            pltpu.sync_copy(table_hbm, table_sh)

        pltpu.sync_copy(idx_hbm.at[wid], idx_v)
        plsc.subcore_barrier()

        def g_start(c, b):
            pltpu.make_async_copy(
                table_sh.at[idx_v.at[c]], rows.at[b], gsem[b]).start()

        def g_wait(b):
            pltpu.make_async_copy(
                table_sh.at[idx_v.at[0]], rows.at[b], gsem[b]).wait()

        def st_start(c, b):
            pltpu.make_async_copy(
                rows.at[b], out_hbm.at[pl.ds(base + c * _C, _C)],
                ssem[b]).start()

        def st_wait(b):
            pltpu.make_async_copy(
                rows.at[b], out_hbm.at[pl.ds(base, _C)], ssem[b]).wait()

        for b in range(4):
            g_start(b, b)

        def body(ii, carry):
            c0 = ii * 4
            for b in range(4):
                g_wait(b)
                st_start(c0 + b, b)
            for b in range(4):
                @pl.when(c0 + 4 + b < npw)
                def _(b=b):
                    st_wait(b)
                    g_start(c0 + 4 + b, b)
            return carry

        lax.fori_loop(0, npw // 4, body, 0)
        for b in range(4):
            st_wait(b)

    return gather


def kernel(positions, row_table, col_table):
    s0, s1 = positions.shape
    b = s0 * s1
    row_p = jnp.pad(row_table.astype(jnp.float32), ((0, 2), (0, 0)))
    col_p = jnp.pad(col_table.astype(jnp.float32), ((0, 2), (0, 0)))
    table = _build_table(row_p, col_p)
    idx = positions.astype(jnp.int32).reshape(_NW, b // (_NW * _C), _C)
    out = _make_gather(b)(table, idx)
    return out.reshape(s0, s1, _D)


# speedup vs baseline: 5.2436x; 1.0002x over previous
import functools

import jax
import jax.numpy as jnp
from jax import lax
from jax.experimental import pallas as pl
from jax.experimental.pallas import tpu as pltpu
from jax.experimental.pallas import tpu_sc as plsc

_D = 128
_HALF = _D // 2
_TPAD = 1024
_NC, _NS = 2, 16
_NW = _NC * _NS
_C = 128


def _table_body(row_ref, col_ref, out_ref):
    t = lax.broadcasted_iota(jnp.int32, (_TPAD, 32), 0)
    k = lax.broadcasted_iota(jnp.int32, (_TPAD, 32), 1)
    r = jnp.clip(t // 30, 0, 29)
    c = t - 30 * (t // 30)
    oh_r = (r == k).astype(jnp.float32)
    oh_c = (c == k).astype(jnp.float32)
    row_emb = jnp.dot(oh_r, row_ref[...], preferred_element_type=jnp.float32,
                      precision=lax.Precision.HIGHEST)
    col_emb = jnp.dot(oh_c, col_ref[...], preferred_element_type=jnp.float32,
                      precision=lax.Precision.HIGHEST)
    out_ref[...] = jnp.concatenate([row_emb, col_emb], axis=-1)


_build_table = pl.pallas_call(
    _table_body,
    out_shape=jax.ShapeDtypeStruct((_TPAD, _D), jnp.float32),
)


@functools.lru_cache(maxsize=None)
def _make_gather(B):
    npw = B // (_NW * _C)
    mesh = plsc.VectorSubcoreMesh(core_axis_name="c", subcore_axis_name="s")

    @functools.partial(
        pl.kernel,
        mesh=mesh,
        out_type=jax.ShapeDtypeStruct((B, _D), jnp.float32),
        scratch_types=[
            pltpu.VMEM((npw, _C), jnp.int32),
            pltpu.VMEM((4 * _C, _D), jnp.float32),
            pltpu.VMEM_SHARED((_TPAD, _D), jnp.float32),
            pltpu.SemaphoreType.DMA,
            pltpu.SemaphoreType.DMA,
            pltpu.SemaphoreType.DMA,
            pltpu.SemaphoreType.DMA,
            pltpu.SemaphoreType.DMA,
            pltpu.SemaphoreType.DMA,
        ],
    )
    def gather(table_hbm, idx_hbm, out_hbm, idx_v, rows, table_sh,
               g0, g1, g2, g3, s0, s1):
        gsem = (g0, g1, g2, g3)
        ssem = (s0, s1)
        wid = lax.axis_index("s") * _NC + lax.axis_index("c")
        base = wid * (npw * _C)

        @pl.when(lax.axis_index("s") == 0)
        def _():
            pltpu.sync_copy(table_hbm, table_sh)

        pltpu.sync_copy(idx_hbm.at[wid], idx_v)
        plsc.subcore_barrier()

        def g_start(c, b):
            pltpu.make_async_copy(
                table_sh.at[idx_v.at[c]],
                rows.at[pl.ds(b * _C, _C)], gsem[b]).start()

        def g_wait(b):
            pltpu.make_async_copy(
                table_sh.at[idx_v.at[0]],
                rows.at[pl.ds(b * _C, _C)], gsem[b]).wait()

        def st_start(c0, h):
            pltpu.make_async_copy(
                rows.at[pl.ds(h * 2 * _C, 2 * _C)],
                out_hbm.at[pl.ds(base + c0 * _C, 2 * _C)], ssem[h]).start()

        def st_wait(h):
            pltpu.make_async_copy(
                rows.at[pl.ds(h * 2 * _C, 2 * _C)],
                out_hbm.at[pl.ds(base, 2 * _C)], ssem[h]).wait()

        for b in range(4):
            g_start(b, b)

        def step(c0, refill):
            for h in range(2):
                g_wait(2 * h)
                g_wait(2 * h + 1)
                st_start(c0 + 2 * h, h)
            if refill:
                for h in range(2):
                    st_wait(h)
                    g_start(c0 + 4 + 2 * h, 2 * h)
                    g_start(c0 + 5 + 2 * h, 2 * h + 1)

        def body(ii, carry):
            step(ii * 4, True)
            return carry

        lax.fori_loop(0, npw // 4 - 1, body, 0)
        step(npw - 4, False)
        for h in range(2):
            st_wait(h)

    return gather


def kernel(positions, row_table, col_table):
    s0, s1 = positions.shape
    b = s0 * s1
    row_p = jnp.pad(row_table.astype(jnp.float32), ((0, 2), (0, 0)))
    col_p = jnp.pad(col_table.astype(jnp.float32), ((0, 2), (0, 0)))
    table = _build_table(row_p, col_p)
    idx = positions.astype(jnp.int32).reshape(_NW, b // (_NW * _C), _C)
    out = _make_gather(b)(table, idx)
    return out.reshape(s0, s1, _D)
